# baseline (speedup 1.0000x reference)
import functools

import jax
import jax.numpy as jnp
from jax import lax
from jax.experimental import pallas as pl
from jax.experimental.pallas import tpu as pltpu
from jax.experimental.pallas import tpu_sc as plsc

_LANES = 16


@functools.lru_cache(maxsize=None)
def _build(B, S, V, D):
    info = plsc.get_sparse_core_info()
    NC, NS = info.num_cores, info.num_subcores
    NW = NC * NS
    N = B * S
    assert N % NW == 0
    b_per_w = N // NW
    R = 32
    while b_per_w % R != 0:
        R //= 2
    n_chunks = b_per_w // R
    assert S % b_per_w == 0
    assert D % _LANES == 0

    mesh = plsc.VectorSubcoreMesh(core_axis_name="c", subcore_axis_name="s")

    @functools.partial(
        pl.kernel,
        out_type=jax.ShapeDtypeStruct((N, D), jnp.float32),
        mesh=mesh,
        scratch_types=[
            pltpu.VMEM((R,), jnp.int32),
            pltpu.VMEM((R, D), jnp.float32),
            pltpu.VMEM((R, D), jnp.float32),
            pltpu.SemaphoreType.DMA,
        ],
    )
    def k(x_hbm, tok_hbm, pe_hbm, out_hbm, idx_c, buf, gbuf, sem):
        wid = lax.axis_index("s") * NC + lax.axis_index("c")
        base = wid * b_per_w
        t0 = lax.rem(base, S)

        def chunk(c, carry):
            off = base + c * R
            toff = t0 + c * R
            pltpu.sync_copy(x_hbm.at[pl.ds(off, R)], idx_c)
            pltpu.sync_copy(pe_hbm.at[pl.ds(toff, R)], buf)
            pltpu.async_copy(tok_hbm.at[idx_c], gbuf, sem).wait()

            def row(r, carry2):
                for j in range(D // _LANES):
                    g = gbuf[r, pl.ds(j * _LANES, _LANES)]
                    plsc.addupdate(buf.at[r, pl.ds(j * _LANES, _LANES)], g)
                return carry2

            lax.fori_loop(0, R, row, 0, unroll=False)
            pltpu.sync_copy(buf, out_hbm.at[pl.ds(off, R)])
            return carry

        lax.fori_loop(0, n_chunks, chunk, 0, unroll=False)

    return k


def kernel(x, tok_emb, pe):
    B, S = x.shape
    V, D = tok_emb.shape
    xf = x.reshape(-1).astype(jnp.int32)
    pe_s = pe[:S, :]
    out = _build(B, S, V, D)(xf, tok_emb, pe_s)
    return out.reshape(B, S, D)


# speedup vs baseline: 1.0421x; 1.0421x over previous
import functools

import jax
import jax.numpy as jnp
from jax import lax
from jax.experimental import pallas as pl
from jax.experimental.pallas import tpu as pltpu
from jax.experimental.pallas import tpu_sc as plsc

_LANES = 16
_NBUF = 3


def _chunk_rows(b_per_w):
    R = 16
    while b_per_w % R != 0:
        R //= 2
    return R


@functools.lru_cache(maxsize=None)
def _build(B, S, V, D):
    info = plsc.get_sparse_core_info()
    NC, NS = info.num_cores, info.num_subcores
    NW = NC * NS
    N = B * S
    assert N % NW == 0
    b_per_w = N // NW
    R = _chunk_rows(b_per_w)
    n_chunks = b_per_w // R
    assert n_chunks >= _NBUF
    assert S % b_per_w == 0
    assert D % _LANES == 0

    mesh = plsc.VectorSubcoreMesh(core_axis_name="c", subcore_axis_name="s")

    scratch = (
        [pltpu.VMEM((n_chunks, R), jnp.int32)]
        + [pltpu.VMEM((R, D), jnp.float32)] * (2 * _NBUF)
        + [pltpu.SemaphoreType.DMA] * (3 * _NBUF)
    )

    @functools.partial(
        pl.kernel,
        out_type=jax.ShapeDtypeStruct((N, D), jnp.float32),
        mesh=mesh,
        scratch_types=scratch,
    )
    def k(x_hbm, tok_hbm, pe_hbm, out_hbm, idx_v, *rest):
        pbufs = rest[0:_NBUF]
        gbufs = rest[_NBUF:2 * _NBUF]
        sems = rest[2 * _NBUF:]
        gsems = sems[0:_NBUF]
        psems = sems[_NBUF:2 * _NBUF]
        ssems = sems[2 * _NBUF:3 * _NBUF]

        wid = lax.axis_index("s") * NC + lax.axis_index("c")
        base = wid * b_per_w
        t0 = lax.rem(base, S)
        pltpu.sync_copy(x_hbm.at[pl.ds(wid * n_chunks, n_chunks)], idx_v)

        loads = {}
        stores = {}

        def issue(c):
            b = c % _NBUF
            g = pltpu.make_async_copy(tok_hbm.at[idx_v.at[c]], gbufs[b],
                                      gsems[b])
            p = pltpu.make_async_copy(pe_hbm.at[pl.ds(t0 + c * R, R)],
                                      pbufs[b], psems[b])
            g.start()
            p.start()
            loads[c] = (g, p)

        issue(0)
        issue(1)

        for c in range(n_chunks):
            b = c % _NBUF
            g, p = loads.pop(c)
            g.wait()
            p.wait()

            def row(r, carry, _b=b):
                for j in range(D // _LANES):
                    v = pbufs[_b][r, pl.ds(j * _LANES, _LANES)]
                    plsc.addupdate(
                        gbufs[_b].at[r, pl.ds(j * _LANES, _LANES)], v)
                return carry

            lax.fori_loop(0, R, row, 0, unroll=False)
            st = pltpu.make_async_copy(
                gbufs[b], out_hbm.at[pl.ds(base + c * R, R)], ssems[b])
            st.start()
            stores[c] = st
            if c + 2 < n_chunks:
                if c - 1 >= 0:
                    stores.pop(c - 1).wait()
                issue(c + 2)

        for c in sorted(stores):
            stores.pop(c).wait()

    return k


def kernel(x, tok_emb, pe):
    B, S = x.shape
    V, D = tok_emb.shape
    xf = x.reshape(-1).astype(jnp.int32)
    pe_s = pe[:S, :]
    info = plsc.get_sparse_core_info()
    R = _chunk_rows(xf.shape[0] // (info.num_cores * info.num_subcores))
    x2 = xf.reshape(-1, R)
    out = _build(B, S, V, D)(x2, tok_emb, pe_s)
    return out.reshape(B, S, D)


# speedup vs baseline: 1.6146x; 1.5494x over previous
import functools

import jax
import jax.numpy as jnp
from jax import lax
from jax.experimental import pallas as pl
from jax.experimental.pallas import tpu as pltpu
from jax.experimental.pallas import tpu_sc as plsc

_LANES = 16
_NBUF = 3


def _grid(B, S):
    info = plsc.get_sparse_core_info()
    NC, NS = info.num_cores, info.num_subcores
    NW = NC * NS
    N = B * S
    assert N % NW == 0
    b_per_w = N // NW
    assert b_per_w % B == 0
    tp = b_per_w // B
    R = 16
    while b_per_w % R != 0 or R % B != 0:
        R //= 2
    assert R % B == 0
    Rq = R // B
    n_chunks = b_per_w // R
    return NC, NS, NW, b_per_w, tp, R, Rq, n_chunks


@functools.lru_cache(maxsize=None)
def _build(B, S, V, D):
    NC, NS, NW, b_per_w, tp, R, Rq, n_chunks = _grid(B, S)
    assert n_chunks >= _NBUF
    assert D % _LANES == 0

    mesh = plsc.VectorSubcoreMesh(core_axis_name="c", subcore_axis_name="s")

    scratch = (
        [pltpu.VMEM((n_chunks, R), jnp.int32),
         pltpu.VMEM((tp, D), jnp.float32)]
        + [pltpu.VMEM((R, D), jnp.float32)] * _NBUF
        + [pltpu.SemaphoreType.DMA] * (2 * _NBUF + 1)
    )

    @functools.partial(
        pl.kernel,
        out_type=jax.ShapeDtypeStruct((B * S, D), jnp.float32),
        mesh=mesh,
        scratch_types=scratch,
    )
    def k(x_hbm, tok_hbm, pe_hbm, out_hbm, idx_v, pbuf, *rest):
        gbufs = rest[0:_NBUF]
        gsems = rest[_NBUF:2 * _NBUF]
        ssems = rest[2 * _NBUF:3 * _NBUF]
        psem = rest[3 * _NBUF]

        wid = lax.axis_index("s") * NC + lax.axis_index("c")
        t0 = wid * tp
        pltpu.sync_copy(x_hbm.at[pl.ds(wid * n_chunks, n_chunks)], idx_v)
        pe_load = pltpu.make_async_copy(pe_hbm.at[pl.ds(t0, tp)], pbuf, psem)
        pe_load.start()

        loads = {}
        stores = {}

        def issue(c):
            b = c % _NBUF
            g = pltpu.make_async_copy(tok_hbm.at[idx_v.at[c]], gbufs[b],
                                      gsems[b])
            g.start()
            loads[c] = g

        for c in range(min(2, n_chunks)):
            issue(c)
        pe_load.wait()

        for c in range(n_chunks):
            rb = c % _NBUF
            loads.pop(c).wait()

            def addj(j, carry, _rb=rb, _c=c):
                off = j * _LANES
                for q in range(Rq):
                    v = pbuf[_c * Rq + q, pl.ds(off, _LANES)]
                    for bb in range(B):
                        plsc.addupdate(
                            gbufs[_rb].at[bb * Rq + q, pl.ds(off, _LANES)], v)
                return carry

            lax.fori_loop(0, D // _LANES, addj, 0, unroll=False)

            sts = []
            for bb in range(B):
                st = pltpu.make_async_copy(
                    gbufs[rb].at[pl.ds(bb * Rq, Rq)],
                    out_hbm.at[pl.ds(bb * S + t0 + c * Rq, Rq)],
                    ssems[rb])
                st.start()
                sts.append(st)
            stores[c] = sts
            if c + 2 < n_chunks:
                if c - 1 >= 0:
                    for st in stores.pop(c - 1):
                        st.wait()
                issue(c + 2)

        for c in sorted(stores):
            for st in stores.pop(c):
                st.wait()

    return k


def kernel(x, tok_emb, pe):
    B, S = x.shape
    V, D = tok_emb.shape
    NC, NS, NW, b_per_w, tp, R, Rq, n_chunks = _grid(B, S)
    x_perm = (x.astype(jnp.int32)
              .reshape(B, NW, n_chunks, Rq)
              .transpose(1, 2, 0, 3)
              .reshape(NW * n_chunks, R))
    pe_s = pe[:S, :]
    out = _build(B, S, V, D)(x_perm, tok_emb, pe_s)
    return out.reshape(B, S, D)


# speedup vs baseline: 1.7054x; 1.0562x over previous
import functools

import jax
import jax.numpy as jnp
from jax import lax
from jax.experimental import pallas as pl
from jax.experimental.pallas import tpu as pltpu
from jax.experimental.pallas import tpu_sc as plsc

_LANES = 16
_NBUF = 5


def _grid(B, S):
    info = plsc.get_sparse_core_info()
    NC, NS = info.num_cores, info.num_subcores
    NW = NC * NS
    N = B * S
    assert N % NW == 0
    b_per_w = N // NW
    assert b_per_w % B == 0
    tp = b_per_w // B
    R = 16
    while b_per_w % R != 0 or R % B != 0:
        R //= 2
    assert R % B == 0
    Rq = R // B
    n_chunks = b_per_w // R
    return NC, NS, NW, b_per_w, tp, R, Rq, n_chunks


@functools.lru_cache(maxsize=None)
def _build(B, S, V, D):
    NC, NS, NW, b_per_w, tp, R, Rq, n_chunks = _grid(B, S)
    assert n_chunks >= _NBUF
    assert D % _LANES == 0

    mesh = plsc.VectorSubcoreMesh(core_axis_name="c", subcore_axis_name="s")

    scratch = (
        [pltpu.VMEM((n_chunks, R), jnp.int32)]
        + [pltpu.VMEM((R, D), jnp.float32)] * _NBUF
        + [pltpu.VMEM((Rq, D), jnp.float32)] * _NBUF
        + [pltpu.SemaphoreType.DMA] * (3 * _NBUF)
    )

    @functools.partial(
        pl.kernel,
        out_type=jax.ShapeDtypeStruct((B * S, D), jnp.float32),
        mesh=mesh,
        scratch_types=scratch,
    )
    def k(x_hbm, tok_hbm, pe_hbm, out_hbm, idx_v, *rest):
        gbufs = rest[0:_NBUF]
        pbufs = rest[_NBUF:2 * _NBUF]
        gsems = rest[2 * _NBUF:3 * _NBUF]
        psems = rest[3 * _NBUF:4 * _NBUF]
        ssems = rest[4 * _NBUF:5 * _NBUF]

        wid = lax.axis_index("s") * NC + lax.axis_index("c")
        t0 = wid * tp
        pltpu.sync_copy(x_hbm.at[pl.ds(wid * n_chunks, n_chunks)], idx_v)

        loads = {}
        stores = {}

        def issue(c):
            b = c % _NBUF
            g = pltpu.make_async_copy(tok_hbm.at[idx_v.at[c]], gbufs[b],
                                      gsems[b])
            p = pltpu.make_async_copy(pe_hbm.at[pl.ds(t0 + c * Rq, Rq)],
                                      pbufs[b], psems[b])
            g.start()
            p.start()
            loads[c] = (g, p)

        for c in range(min(_NBUF - 1, n_chunks)):
            issue(c)

        for c in range(n_chunks):
            rb = c % _NBUF
            g, p = loads.pop(c)
            g.wait()
            p.wait()

            def addj(j, carry, _rb=rb):
                off = j * _LANES
                for q in range(Rq):
                    v = pbufs[_rb][q, pl.ds(off, _LANES)]
                    for bb in range(B):
                        plsc.addupdate(
                            gbufs[_rb].at[bb * Rq + q, pl.ds(off, _LANES)], v)
                return carry

            lax.fori_loop(0, D // _LANES, addj, 0, unroll=False)

            sts = []
            for bb in range(B):
                st = pltpu.make_async_copy(
                    gbufs[rb].at[pl.ds(bb * Rq, Rq)],
                    out_hbm.at[pl.ds(bb * S + t0 + c * Rq, Rq)],
                    ssems[rb])
                st.start()
                sts.append(st)
            stores[c] = sts
            nxt = c + _NBUF - 1
            if nxt < n_chunks:
                if c >= 1:
                    for st in stores.pop(c - 1):
                        st.wait()
                issue(nxt)

        for c in sorted(stores):
            for st in stores.pop(c):
                st.wait()

    return k


def kernel(x, tok_emb, pe):
    B, S = x.shape
    V, D = tok_emb.shape
    NC, NS, NW, b_per_w, tp, R, Rq, n_chunks = _grid(B, S)
    x_perm = (x.astype(jnp.int32)
              .reshape(B, NW, n_chunks, Rq)
              .transpose(1, 2, 0, 3)
              .reshape(NW * n_chunks, R))
    pe_s = pe[:S, :]
    out = _build(B, S, V, D)(x_perm, tok_emb, pe_s)
    return out.reshape(B, S, D)


# speedup vs baseline: 1.7304x; 1.0147x over previous
import functools

import jax
import jax.numpy as jnp
from jax import lax
from jax.experimental import pallas as pl
from jax.experimental.pallas import tpu as pltpu
from jax.experimental.pallas import tpu_sc as plsc

_LANES = 16
_GBUF = 4
_PBUF = 2


def _grid(B, S):
    info = plsc.get_sparse_core_info()
    NC, NS = info.num_cores, info.num_subcores
    NW = NC * NS
    N = B * S
    assert N % NW == 0
    b_per_w = N // NW
    assert b_per_w % B == 0
    tp = b_per_w // B
    Rc = 16
    while tp % Rc != 0:
        Rc //= 2
    n_tc = tp // Rc
    return NC, NS, NW, b_per_w, tp, Rc, n_tc


@functools.lru_cache(maxsize=None)
def _build(B, S, V, D):
    NC, NS, NW, b_per_w, tp, Rc, n_tc = _grid(B, S)
    NU = B * n_tc
    assert NU >= _GBUF and n_tc >= _PBUF
    assert D % _LANES == 0

    mesh = plsc.VectorSubcoreMesh(core_axis_name="c", subcore_axis_name="s")

    scratch = (
        [pltpu.VMEM((B, n_tc, Rc), jnp.int32)]
        + [pltpu.VMEM((Rc, D), jnp.float32)] * (_GBUF + _PBUF)
        + [pltpu.SemaphoreType.DMA] * (2 * _GBUF + _PBUF + 1)
    )

    @functools.partial(
        pl.kernel,
        out_type=jax.ShapeDtypeStruct((B * S, D), jnp.float32),
        mesh=mesh,
        scratch_types=scratch,
    )
    def k(x_hbm, tok_hbm, pe_hbm, out_hbm, idx_v, *rest):
        gbufs = rest[0:_GBUF]
        pbufs = rest[_GBUF:_GBUF + _PBUF]
        sems = rest[_GBUF + _PBUF:]
        gsems = sems[0:_GBUF]
        ssems = sems[_GBUF:2 * _GBUF]
        psems = sems[2 * _GBUF:2 * _GBUF + _PBUF]
        isem = sems[2 * _GBUF + _PBUF]

        wid = lax.axis_index("s") * NC + lax.axis_index("c")
        t0 = wid * tp
        rows_per_b = S // Rc
        for bb in range(B):
            pltpu.make_async_copy(
                x_hbm.at[pl.ds(bb * rows_per_b + wid * n_tc, n_tc)],
                idx_v.at[bb], isem).start()
        for bb in range(B):
            pltpu.make_async_copy(
                x_hbm.at[pl.ds(bb * rows_per_b + wid * n_tc, n_tc)],
                idx_v.at[bb], isem).wait()

        pe_loads = {}
        loads = {}
        stores = {}

        def issue_pe(cc):
            pb = cc % _PBUF
            p = pltpu.make_async_copy(
                pe_hbm.at[pl.ds(t0 + cc * Rc, Rc)], pbufs[pb], psems[pb])
            p.start()
            pe_loads[cc] = p

        def issue_gather(u):
            gb = u % _GBUF
            cc, bb = divmod(u, B)
            g = pltpu.make_async_copy(
                tok_hbm.at[idx_v.at[bb, cc]], gbufs[gb], gsems[gb])
            g.start()
            loads[u] = g

        for cc in range(min(_PBUF, n_tc)):
            issue_pe(cc)
        for u in range(min(_GBUF - 1, NU)):
            issue_gather(u)

        for u in range(NU):
            cc, bb = divmod(u, B)
            gb = u % _GBUF
            pb = cc % _PBUF
            if bb == 0:
                pe_loads.pop(cc).wait()
            loads.pop(u).wait()

            def addj(j, carry, _gb=gb, _pb=pb):
                off = j * _LANES
                for r in range(Rc):
                    v = pbufs[_pb][r, pl.ds(off, _LANES)]
                    plsc.addupdate(
                        gbufs[_gb].at[r, pl.ds(off, _LANES)], v)
                return carry

            lax.fori_loop(0, D // _LANES, addj, 0, unroll=False)

            st = pltpu.make_async_copy(
                gbufs[gb],
                out_hbm.at[pl.ds(bb * S + t0 + cc * Rc, Rc)],
                ssems[gb])
            st.start()
            stores[u] = st

            if bb == B - 1 and cc + _PBUF < n_tc:
                issue_pe(cc + _PBUF)
            nxt = u + _GBUF - 1
            if nxt < NU:
                if u >= 1:
                    stores.pop(u - 1).wait()
                issue_gather(nxt)

        for u in sorted(stores):
            stores.pop(u).wait()

    return k


def kernel(x, tok_emb, pe):
    B, S = x.shape
    V, D = tok_emb.shape
    NC, NS, NW, b_per_w, tp, Rc, n_tc = _grid(B, S)
    x2 = x.astype(jnp.int32).reshape(B * S // Rc, Rc)
    pe_s = pe[:S, :]
    out = _build(B, S, V, D)(x2, tok_emb, pe_s)
    return out.reshape(B, S, D)
